# baseline (speedup 1.0000x reference)
import functools

import jax
import jax.numpy as jnp
from jax import lax
from jax.experimental import pallas as pl
from jax.experimental.pallas import tpu as pltpu
from jax.experimental.pallas import tpu_sc as plsc

_B = 16384
_D = 26
_K = 1000
_HB = _B // 2
_UB = 1024
_SPLIT = _HB // _UB
_NU = _SPLIT * _D
_NW = 32


def _sc_gather(dwt2d, mean_flat, lv_flat, labels_t, h):
    mesh = plsc.VectorSubcoreMesh(core_axis_name="c", subcore_axis_name="s")

    @functools.partial(
        pl.kernel,
        out_type=[
            jax.ShapeDtypeStruct((_D, _D, _HB), jnp.float32),
            jax.ShapeDtypeStruct((_D, _HB), jnp.float32),
            jax.ShapeDtypeStruct((_D, _HB), jnp.float32),
        ],
        mesh=mesh,
        compiler_params=pltpu.CompilerParams(
            needs_layout_passes=False, use_tc_tiling_on_sc=False),
        scratch_types=(
            [pltpu.VMEM((_D, _K), jnp.float32)]
            + [pltpu.VMEM((_K,), jnp.float32)] * 2
            + [pltpu.VMEM((_UB,), jnp.int32)] * 2
            + [pltpu.VMEM((32,), jnp.int32)]
            + [pltpu.VMEM((_D, 1, _UB), jnp.float32)] * 2
            + [pltpu.VMEM((1, _UB), jnp.float32)] * 2
            + [pltpu.VMEM((1, _UB), jnp.float32)] * 2
            + [pltpu.SemaphoreType.DMA] * 5
        ),
    )
    def k(dwt_hbm, mean_hbm, lv_hbm, labt_hbm,
          dw_hbm, mt_hbm, lt_hbm,
          rows, mrow, lrow, lb0, lb1, ridx,
          ob0, ob1, om0, om1, ol0, ol1,
          sgat, sla0, sla1, so0, so1):
        lbuf = (lb0, lb1)
        slab = (sla0, sla1)
        obdw = (ob0, ob1)
        obm = (om0, om1)
        obl = (ol0, ol1)
        sout = (so0, so1)

        wid = lax.axis_index("c") * 16 + lax.axis_index("s")
        lanes = lax.iota(jnp.int32, 16)
        splat_i = [jnp.full((16,), i, jnp.int32) for i in range(_D)]

        u_start = (_NU * wid) // _NW
        u_end = (_NU * (wid + 1)) // _NW

        def lab_slice(u):
            j = u // _SPLIT
            sp = u - _SPLIT * j
            return labt_hbm.at[pl.ds(j * _B + h * _HB + sp * _UB, _UB)]

        def fire_labels(u, p):
            pltpu.async_copy(lab_slice(u), lbuf[p], slab[p])

        def wait_labels(u, p):
            pltpu.make_async_copy(lab_slice(u), lbuf[p], slab[p]).wait()

        def stage_rows(j):
            ridx[pl.ds(0, 16)] = lanes * _D + j
            ridx[pl.ds(16, 16)] = (lanes + 16) * _D + j
            pltpu.async_copy(dwt_hbm.at[ridx.at[pl.ds(0, _D)]], rows, sgat)
            pltpu.sync_copy(mean_hbm.at[pl.ds(j * _K, _K)], mrow)
            pltpu.sync_copy(lv_hbm.at[pl.ds(j * _K, _K)], lrow)
            pltpu.make_async_copy(
                dwt_hbm.at[ridx.at[pl.ds(0, _D)]], rows, sgat).wait()

        def out_slices(j, sp):
            b0 = sp * _UB
            return (dw_hbm.at[:, pl.ds(j, 1), pl.ds(b0, _UB)],
                    mt_hbm.at[pl.ds(j, 1), pl.ds(b0, _UB)],
                    lt_hbm.at[pl.ds(j, 1), pl.ds(b0, _UB)])

        def fire_out(j, sp, s):
            dws, ms, ls = out_slices(j, sp)
            pltpu.async_copy(obdw[s], dws, sout[s])
            pltpu.async_copy(obm[s], ms, sout[s])
            pltpu.async_copy(obl[s], ls, sout[s])

        def wait_out(j, sp, s):
            dws, ms, ls = out_slices(j, sp)
            pltpu.make_async_copy(obdw[s], dws, sout[s]).wait()
            pltpu.make_async_copy(obm[s], ms, sout[s]).wait()
            pltpu.make_async_copy(obl[s], ls, sout[s]).wait()

        fire_labels(u_start, 0)

        def halfunit(u, lp, prev_j, first):
            j = u // _SPLIT
            sp = u - _SPLIT * j

            @pl.when(j != prev_j)
            def _():
                stage_rows(j)
            wait_labels(u, lp)

            @pl.when(u + 1 < u_end)
            def _():
                fire_labels(u + 1, 1 - lp)

            @pl.when(jnp.logical_not(first))
            def _():
                wait_out(j, sp, lp)

            @plsc.parallel_loop(0, _UB // 16, unroll=2)
            def v_body(v):
                idxv = lbuf[lp][pl.ds(v * 16, 16)]
                for i in range(_D):
                    val = plsc.load_gather(rows, [splat_i[i], idxv])
                    obdw[lp][i, 0, pl.ds(v * 16, 16)] = val
                obm[lp][0, pl.ds(v * 16, 16)] = plsc.load_gather(mrow, [idxv])
                obl[lp][0, pl.ds(v * 16, 16)] = plsc.load_gather(lrow, [idxv])
            fire_out(j, sp, lp)

        def pairbody(gg, prev_j):
            u = u_start + 2 * gg
            halfunit(u, 0, prev_j, gg == 0)
            j0 = u // _SPLIT

            @pl.when(u + 1 < u_end)
            def _():
                halfunit(u + 1, 1, j0, gg == 0)
            j1 = (u + 1) // _SPLIT
            return jnp.where(u + 1 < u_end, j1, j0)
        lax.fori_loop(0, (u_end - u_start + 1) // 2, pairbody,
                      jnp.int32(-1))

        u_last = u_end - 1
        j = u_last // _SPLIT
        sp = u_last - _SPLIT * j
        wait_out(j, sp, 0)
        wait_out(j, sp, 1)

    return k(dwt2d, mean_flat, lv_flat, labels_t)


def kernel(labels, mean, log_var, domain_weights):
    labels = labels.astype(jnp.int32)
    labels_t = jnp.transpose(labels).reshape(-1)
    dwt2d = domain_weights.reshape(_D * _D, _K)
    mean_flat = mean.reshape(-1)
    lv_flat = log_var.reshape(-1)
    parts = [_sc_gather(dwt2d, mean_flat, lv_flat, labels_t, h)
             for h in (0, 1)]
    dwp = jnp.concatenate([p[0] for p in parts], axis=2)
    mt = jnp.concatenate([p[1] for p in parts], axis=1)
    lt = jnp.concatenate([p[2] for p in parts], axis=1)
    means = jnp.transpose(mt)
    log_vars = jnp.transpose(lt)
    dw = jnp.transpose(dwp, (2, 0, 1))
    return (means, log_vars, dw)


# speedup vs baseline: 1.4914x; 1.4914x over previous
import functools

import jax
import jax.numpy as jnp
from jax import lax
from jax.experimental import pallas as pl
from jax.experimental.pallas import tpu as pltpu
from jax.experimental.pallas import tpu_sc as plsc

_B = 16384
_D = 26
_K = 1000
_SPLIT = 16
_NU = _SPLIT * _D
_UB = _B // _SPLIT
_CH = _UB
_NW = 32


def _sc_gather(dwt2d, mean_flat, lv_flat, labels_t):
    mesh = plsc.VectorSubcoreMesh(core_axis_name="c", subcore_axis_name="s")

    @functools.partial(
        pl.kernel,
        out_type=[
            jax.ShapeDtypeStruct((_D, _D, _B), jnp.float32),
            jax.ShapeDtypeStruct((_D, _B), jnp.float32),
            jax.ShapeDtypeStruct((_D, _B), jnp.float32),
        ],
        mesh=mesh,
        compiler_params=pltpu.CompilerParams(
            needs_layout_passes=False, use_tc_tiling_on_sc=False),
        scratch_types=(
            [pltpu.VMEM((_D, _K), jnp.float32)]
            + [pltpu.VMEM((_K,), jnp.float32)] * 2
            + [pltpu.VMEM((_UB,), jnp.int32)] * 2
            + [pltpu.VMEM((32,), jnp.int32)]
            + [pltpu.VMEM((_D, 1, _CH), jnp.float32)] * 2
            + [pltpu.VMEM((1, _CH), jnp.float32)] * 2
            + [pltpu.VMEM((1, _CH), jnp.float32)] * 2
            + [pltpu.SemaphoreType.DMA] * 5
        ),
    )
    def k(dwt_hbm, mean_hbm, lv_hbm, labt_hbm,
          dw_hbm, mt_hbm, lt_hbm,
          rows, mrow, lrow, lb0, lb1, ridx,
          ob0, ob1, om0, om1, ol0, ol1,
          sgat, sla0, sla1, so0, so1):
        lbuf = (lb0, lb1)
        slab = (sla0, sla1)
        obdw = (ob0, ob1)
        obm = (om0, om1)
        obl = (ol0, ol1)
        sout = (so0, so1)

        wid = lax.axis_index("c") * 16 + lax.axis_index("s")
        lanes = lax.iota(jnp.int32, 16)
        splat_i = [jnp.full((16,), i, jnp.int32) for i in range(_D)]

        u_start = (_NU * wid) // _NW
        u_end = (_NU * (wid + 1)) // _NW

        def lab_slice(u):
            j = u // _SPLIT
            sp = u - _SPLIT * j
            return labt_hbm.at[pl.ds(j * _B + sp * _UB, _UB)]

        def fire_labels(u, p):
            pltpu.async_copy(lab_slice(u), lbuf[p], slab[p])

        def wait_labels(u, p):
            pltpu.make_async_copy(lab_slice(u), lbuf[p], slab[p]).wait()

        def stage_rows(j):
            ridx[pl.ds(0, 16)] = lanes * _D + j
            ridx[pl.ds(16, 16)] = (lanes + 16) * _D + j
            pltpu.async_copy(dwt_hbm.at[ridx.at[pl.ds(0, _D)]], rows, sgat)
            pltpu.sync_copy(mean_hbm.at[pl.ds(j * _K, _K)], mrow)
            pltpu.sync_copy(lv_hbm.at[pl.ds(j * _K, _K)], lrow)
            pltpu.make_async_copy(
                dwt_hbm.at[ridx.at[pl.ds(0, _D)]], rows, sgat).wait()

        def out_slices(j, sp, c, s):
            b0 = sp * _UB + c * _CH
            return (dw_hbm.at[:, pl.ds(j, 1), pl.ds(b0, _CH)],
                    mt_hbm.at[pl.ds(j, 1), pl.ds(b0, _CH)],
                    lt_hbm.at[pl.ds(j, 1), pl.ds(b0, _CH)])

        def fire_out(j, sp, c, s):
            dws, ms, ls = out_slices(j, sp, c, s)
            pltpu.async_copy(obdw[s], dws, sout[s])
            pltpu.async_copy(obm[s], ms, sout[s])
            pltpu.async_copy(obl[s], ls, sout[s])

        def wait_out(j, sp, c, s):
            dws, ms, ls = out_slices(j, sp, c, s)
            pltpu.make_async_copy(obdw[s], dws, sout[s]).wait()
            pltpu.make_async_copy(obm[s], ms, sout[s]).wait()
            pltpu.make_async_copy(obl[s], ls, sout[s]).wait()

        fire_labels(u_start, 0)

        def halfunit(u, lp, prev_j, first):
            j = u // _SPLIT
            sp = u - _SPLIT * j

            @pl.when(j != prev_j)
            def _():
                stage_rows(j)
            wait_labels(u, lp)

            @pl.when(u + 1 < u_end)
            def _():
                fire_labels(u + 1, 1 - lp)

            @pl.when(jnp.logical_not(first))
            def _():
                wait_out(j, sp, 0, lp)

            @plsc.parallel_loop(0, _CH // 16, unroll=2)
            def v_body(v):
                idxv = lbuf[lp][pl.ds(v * 16, 16)]
                for i in range(_D):
                    val = plsc.load_gather(rows, [splat_i[i], idxv])
                    obdw[lp][i, 0, pl.ds(v * 16, 16)] = val
                obm[lp][0, pl.ds(v * 16, 16)] = plsc.load_gather(mrow, [idxv])
                obl[lp][0, pl.ds(v * 16, 16)] = plsc.load_gather(lrow, [idxv])
            fire_out(j, sp, 0, lp)

        def pairbody(gg, prev_j):
            u = u_start + 2 * gg
            halfunit(u, 0, prev_j, gg == 0)
            j0 = u // _SPLIT

            @pl.when(u + 1 < u_end)
            def _():
                halfunit(u + 1, 1, j0, gg == 0)
            j1 = (u + 1) // _SPLIT
            return jnp.where(u + 1 < u_end, j1, j0)
        lax.fori_loop(0, (u_end - u_start + 1) // 2, pairbody,
                      jnp.int32(-1))

        u_last = u_end - 1
        j = u_last // _SPLIT
        sp = u_last - _SPLIT * j
        wait_out(j, sp, 0, 0)
        wait_out(j, sp, 0, 1)

    return k(dwt2d, mean_flat, lv_flat, labels_t)


def kernel(labels, mean, log_var, domain_weights):
    labels = labels.astype(jnp.int32)
    labels_t = jnp.transpose(labels).reshape(-1)
    dwp, mt, lt = _sc_gather(
        domain_weights.reshape(_D * _D, _K),
        mean.reshape(-1), log_var.reshape(-1), labels_t)
    means = jnp.transpose(mt)
    log_vars = jnp.transpose(lt)
    dw = jnp.transpose(dwp, (2, 0, 1))
    return (means, log_vars, dw)
